# Initial kernel scaffold; baseline (speedup 1.0000x reference)
import jax
import jax.numpy as jnp
from jax.experimental import pallas as pl


def kernel(x, edge_index, Wl, bl, att, gat_bias, W2, b2):
    raise NotImplementedError("write your pallas kernel here")


# speedup vs baseline: 18.1039x; 18.1039x over previous
import jax
import jax.numpy as jnp
from jax import lax
from jax.experimental import pallas as pl
from jax.experimental.pallas import tpu as pltpu
from jax.experimental.pallas import tpu_sc as plsc

N = 10000
E = 320000
FIN = 128
H = 4
F = 128
HF = H * F
C = 16

NC, NS, L = 2, 16, 16
NW = NC * NS
EPT = E // NW
R = 2176
NPASS = -(-N // R)
RP = R // NS
SSH = 624
SEG = 2000
NSEG = EPT // SEG
K = 32
K2 = 80
NCH2 = EPT // K2
FB = F // 16

_f32 = jnp.float32
_i32 = jnp.int32


def _a_body(x_ref, w_ref, b_ref, o0, o1, o2, o3):
    res = (jnp.dot(x_ref[...], w_ref[...], preferred_element_type=_f32)
           + b_ref[...])
    o0[...] = res[:, 0 * F:1 * F]
    o1[...] = res[:, 1 * F:2 * F]
    o2[...] = res[:, 2 * F:3 * F]
    o3[...] = res[:, 3 * F:4 * F]


def _run_a(x, Wl, bl):
    bn = 1000
    sds = jax.ShapeDtypeStruct((N, F), _f32)
    return pl.pallas_call(
        _a_body,
        grid=(N // bn,),
        in_specs=[
            pl.BlockSpec((bn, FIN), lambda i: (i, 0)),
            pl.BlockSpec((FIN, HF), lambda i: (0, 0)),
            pl.BlockSpec((1, HF), lambda i: (0, 0)),
        ],
        out_specs=[pl.BlockSpec((bn, F), lambda i: (i, 0))] * H,
        out_shape=[sds] * H,
    )(x, Wl, bl.reshape(1, HF))


def _gat_body(src_h, dst_h, xl0, xl1, xl2, xl3, attf, gat_part, s_part,
              ssrc, sdst, csrc, cdst, idxb,
              rs0, rs1, rs2, rs3, rd0, rd1, rd2, rd3, svecb,
              att_v, z16v, s_acc, ga0, ga1, ga2, ga3, sem, sem2):
    cid = lax.axis_index("c")
    sid = lax.axis_index("s")
    wid = cid * NS + sid
    est = wid * EPT
    xls = (xl0, xl1, xl2, xl3)
    rss = (rs0, rs1, rs2, rs3)
    rds = (rd0, rd1, rd2, rd3)
    gas = (ga0, ga1, ga2, ga3)
    pltpu.sync_copy(attf, att_v)

    zv = jnp.zeros((16,), _f32)

    def _zfill(i, carry):
        for b in range(FB):
            rs0[i, pl.ds(b * 16, 16)] = zv
        z16v[i, :] = zv
        return carry

    lax.fori_loop(0, 16, _zfill, 0)

    sbase = sid * SSH
    for j in range(SSH // 16):
        pltpu.sync_copy(z16v, s_acc.at[pl.ds(sbase + j * 16, 16)])

    @pl.when(sid == NS - 1)
    def _():
        pltpu.sync_copy(z16v, s_acc.at[pl.ds(NS * SSH, 16)])

    def _pass(p, pcarry):
        base = pl.multiple_of(p * R, 128)
        gb = sid * RP
        for h in range(H):
            for j in range(RP // 16):
                pltpu.sync_copy(rs0.at[pl.ds(0, 16)],
                                gas[h].at[pl.ds(gb + j * 16, 16)])
        plsc.subcore_barrier()

        def _seg(seg, scarry):
            soff = pl.multiple_of(est + seg * SEG, 8)
            pltpu.sync_copy(src_h.at[pl.ds(soff, SEG)], ssrc)
            pltpu.sync_copy(dst_h.at[pl.ds(soff, SEG)], sdst)

            def _comp(i, cnt):
                s16 = ssrc[pl.ds(i * 16, 16)]
                d16 = sdst[pl.ds(i * 16, 16)]
                m = (d16 >= base) & (d16 < base + R)
                mi = m.astype(_i32)
                pos = cnt + plsc.cumsum(mi) - mi
                plsc.store_scatter(csrc, [pos], s16, mask=m)
                plsc.store_scatter(cdst, [pos], d16, mask=m)
                return cnt + plsc.all_reduce_population_count(m)[0]

            cnt = lax.fori_loop(0, SEG // 16, _comp, jnp.asarray(0, _i32))

            csrc[pl.ds(cnt, 16)] = jnp.zeros((16,), _i32)
            csrc[pl.ds(cnt + 16, 16)] = jnp.zeros((16,), _i32)
            cdst[pl.ds(cnt, 16)] = jnp.full((16,), base, _i32)
            cdst[pl.ds(cnt + 16, 16)] = jnp.full((16,), base, _i32)
            nch = (cnt + (K - 1)) // K

            def _chunk(c, carry):
                off = c * K
                for j in range(K // 16):
                    sv = csrc[pl.ds(off + j * 16, 16)]
                    dv = cdst[pl.ds(off + j * 16, 16)]
                    idxb[0, pl.ds(j * 16, 16)] = sv
                    idxb[1, pl.ds(j * 16, 16)] = dv
                    idxb[2, pl.ds(j * 16, 16)] = dv - base
                cps = []
                for h in range(H):
                    cps.append(pltpu.async_copy(
                        xls[h].at[idxb.at[0]], rss[h], sem))
                    cps.append(pltpu.async_copy(
                        xls[h].at[idxb.at[1]], rds[h], sem2))
                for cp in cps:
                    cp.wait()

                def _edge(e, ecarry):
                    valid = jnp.where(off + e < cnt, 1.0, 0.0).astype(_f32)
                    exs = []
                    for h in range(H):
                        acc = jnp.zeros((16,), _f32)
                        for b in range(FB):
                            z = (rss[h][e, pl.ds(b * 16, 16)]
                                 + rds[h][e, pl.ds(b * 16, 16)])
                            lr = (jnp.maximum(z, 0.0)
                                  + 0.2 * jnp.minimum(z, 0.0))
                            acc = acc + lr * att_v[pl.ds(h * F + b * 16, 16)]
                        sc = jnp.sum(acc)
                        exv = jnp.exp(jnp.full((16,), sc, _f32))
                        exs.append(exv[0] * valid)
                    for h in range(H):
                        for b in range(FB):
                            rss[h][e, pl.ds(b * 16, 16)] = (
                                rss[h][e, pl.ds(b * 16, 16)] * exs[h])
                    lane = lax.iota(_i32, 16)
                    sv = jnp.where(lane == 0, exs[0], 0.0)
                    sv = sv + jnp.where(lane == 1, exs[1], 0.0)
                    sv = sv + jnp.where(lane == 2, exs[2], 0.0)
                    sv = sv + jnp.where(lane == 3, exs[3], 0.0)
                    sv = sv + jnp.where(lane == 4, valid, 0.0)
                    svecb[e, :] = sv
                    return ecarry

                lax.fori_loop(0, K, _edge, 0)
                pltpu.sync_copy(svecb, s_acc.at[idxb.at[1]], add=True)
                for h in range(H):
                    pltpu.sync_copy(rss[h], gas[h].at[idxb.at[2]], add=True)
                return carry

            lax.fori_loop(0, nch, _chunk, 0)
            return scarry

        lax.fori_loop(0, NSEG, _seg, 0)

        lax.fori_loop(0, 16, _zfill, 0)
        plsc.subcore_barrier()

        for h in range(H):
            for j in range(RP // 16):
                pltpu.sync_copy(gas[h].at[pl.ds(gb + j * 16, 16)],
                                rd0.at[pl.ds(0, 16)])
                pltpu.sync_copy(
                    rd0.at[pl.ds(0, 16)],
                    gat_part.at[cid, h, pl.ds(base + gb + j * 16, 16)])
        plsc.subcore_barrier()
        return pcarry

    lax.fori_loop(0, NPASS, _pass, 0)

    for j in range(SSH // 16):
        pltpu.sync_copy(s_acc.at[pl.ds(sbase + j * 16, 16)],
                        svecb.at[pl.ds(0, 16)])
        pltpu.sync_copy(svecb.at[pl.ds(0, 16)],
                        s_part.at[cid, pl.ds(sbase + j * 16, 16)])

    @pl.when(sid == NS - 1)
    def _():
        pltpu.sync_copy(s_acc.at[pl.ds(NS * SSH, 16)],
                        svecb.at[pl.ds(0, 16)])
        pltpu.sync_copy(svecb.at[pl.ds(0, 16)],
                        s_part.at[cid, pl.ds(NS * SSH, 16)])


def _run_gat(src, dst, xls, att_flat):
    mesh = plsc.VectorSubcoreMesh(core_axis_name="c", subcore_axis_name="s",
                                  num_cores=NC, num_subcores=NS)
    f = pl.kernel(
        _gat_body,
        out_type=(
            jax.ShapeDtypeStruct((NC, H, NPASS * R, F), _f32),
            jax.ShapeDtypeStruct((NC, N, 16), _f32),
        ),
        mesh=mesh,
        compiler_params=pltpu.CompilerParams(needs_layout_passes=False),
        scratch_types=[
            pltpu.VMEM((SEG,), _i32),
            pltpu.VMEM((SEG,), _i32),
            pltpu.VMEM((SEG + 64,), _i32),
            pltpu.VMEM((SEG + 64,), _i32),
            pltpu.VMEM((4, K), _i32),
            pltpu.VMEM((K, F), _f32),
            pltpu.VMEM((K, F), _f32),
            pltpu.VMEM((K, F), _f32),
            pltpu.VMEM((K, F), _f32),
            pltpu.VMEM((K, F), _f32),
            pltpu.VMEM((K, F), _f32),
            pltpu.VMEM((K, F), _f32),
            pltpu.VMEM((K, F), _f32),
            pltpu.VMEM((K, 16), _f32),
            pltpu.VMEM((HF,), _f32),
            pltpu.VMEM((16, 16), _f32),
            pltpu.VMEM_SHARED((N, 16), _f32),
            pltpu.VMEM_SHARED((R, F), _f32),
            pltpu.VMEM_SHARED((R, F), _f32),
            pltpu.VMEM_SHARED((R, F), _f32),
            pltpu.VMEM_SHARED((R, F), _f32),
            pltpu.SemaphoreType.DMA,
            pltpu.SemaphoreType.DMA,
        ],
    )
    return f(src, dst, xls[0], xls[1], xls[2], xls[3], att_flat)


def _c_body(x0, x1, x2, x3, p00, p01, p02, p03, p10, p11, p12, p13,
            s0_ref, s1_ref, att_ref, gb_ref, w2_ref, q_ref, lt_ref, db_ref):
    xbs = (x0[...], x1[...], x2[...], x3[...])
    gp0 = (p00, p01, p02, p03)
    gp1 = (p10, p11, p12, p13)
    ssum = s0_ref[...] + s1_ref[...]
    gats = []
    for h in range(H):
        xh = xbs[h]
        t = jnp.where(xh > 0, 2.0 * xh, 0.4 * xh) * att_ref[h, :][None, :]
        ex_h = jnp.exp(jnp.sum(t, axis=1))
        stot = ssum[:, h] + ex_h
        gats.append((gp0[h][...] + gp1[h][...]
                     + ex_h[:, None] * xh) / stot[:, None])
    hcat = jnp.concatenate(gats, axis=1) + gb_ref[...]
    hcat = jnp.where(hcat > 0, hcat,
                     jnp.exp(jnp.minimum(hcat, 0.0)) - 1.0)
    hw = jnp.dot(hcat, w2_ref[...], preferred_element_type=_f32)
    deg = ssum[:, 4] + 1.0
    dis = lax.rsqrt(deg)
    q = dis[:, None] * hw
    q_ref[...] = jnp.concatenate(
        [q, jnp.zeros((q.shape[0], F - C), _f32)], axis=1)
    lt_ref[...] = hw / deg[:, None]
    db_ref[...] = jnp.broadcast_to(dis[:, None], hw.shape)


def _run_c(xls, gps, s0, s1, att, gat_bias, W2):
    bn = 1000
    out_sds = jax.ShapeDtypeStruct((N, C), _f32)
    q_sds = jax.ShapeDtypeStruct((N, F), _f32)
    bF = pl.BlockSpec((bn, F), lambda i: (i, 0))
    b16 = pl.BlockSpec((bn, 16), lambda i: (i, 0))
    return pl.pallas_call(
        _c_body,
        grid=(N // bn,),
        in_specs=(
            [bF] * 4 + [bF] * 8 + [b16, b16]
            + [pl.BlockSpec((H, F), lambda i: (0, 0)),
               pl.BlockSpec((1, HF), lambda i: (0, 0)),
               pl.BlockSpec((HF, C), lambda i: (0, 0))]
        ),
        out_specs=[
            pl.BlockSpec((bn, F), lambda i: (i, 0)),
            pl.BlockSpec((bn, C), lambda i: (i, 0)),
            pl.BlockSpec((bn, C), lambda i: (i, 0)),
        ],
        out_shape=[q_sds, out_sds, out_sds],
    )(*xls, *gps, s0, s1, att, gat_bias.reshape(1, HF), W2)


def _gcn_body(src_h, dst_h, q, out_part, src_v, dst_v, qrows, idxb2, z16v,
              out_acc, sem):
    cid = lax.axis_index("c")
    sid = lax.axis_index("s")
    wid = cid * NS + sid
    est = wid * EPT
    pltpu.sync_copy(src_h.at[pl.ds(est, EPT)], src_v)
    pltpu.sync_copy(dst_h.at[pl.ds(est, EPT)], dst_v)

    zv = jnp.zeros((16,), _f32)

    def _zfill(i, carry):
        for b in range(FB):
            qrows[i, pl.ds(b * 16, 16)] = zv
        return carry

    lax.fori_loop(0, 16, _zfill, 0)
    sbase = sid * SSH
    for j in range(SSH // 16):
        pltpu.sync_copy(qrows.at[pl.ds(0, 16)],
                        out_acc.at[pl.ds(sbase + j * 16, 16)])

    @pl.when(sid == NS - 1)
    def _():
        pltpu.sync_copy(qrows.at[pl.ds(0, 16)],
                        out_acc.at[pl.ds(NS * SSH, 16)])

    plsc.subcore_barrier()

    def _chunk(c, carry):
        off = c * K2
        for j in range(K2 // 16):
            idxb2[0, pl.ds(j * 16, 16)] = src_v[pl.ds(off + j * 16, 16)]
            idxb2[1, pl.ds(j * 16, 16)] = dst_v[pl.ds(off + j * 16, 16)]
        pltpu.async_copy(q.at[idxb2.at[0]], qrows, sem).wait()
        pltpu.sync_copy(qrows, out_acc.at[idxb2.at[1]], add=True)
        return carry

    lax.fori_loop(0, NCH2, _chunk, 0)
    plsc.subcore_barrier()

    for j in range(8):
        rows = 80 if j < 7 else SSH - 7 * 80
        pltpu.sync_copy(out_acc.at[pl.ds(sbase + j * 80, rows)],
                        qrows.at[pl.ds(0, rows)])
        pltpu.sync_copy(qrows.at[pl.ds(0, rows)],
                        out_part.at[cid, pl.ds(sbase + j * 80, rows)])

    @pl.when(sid == NS - 1)
    def _():
        pltpu.sync_copy(out_acc.at[pl.ds(NS * SSH, 16)],
                        qrows.at[pl.ds(0, 16)])
        pltpu.sync_copy(qrows.at[pl.ds(0, 16)],
                        out_part.at[cid, pl.ds(NS * SSH, 16)])


def _run_gcn(src, dst, q):
    mesh = plsc.VectorSubcoreMesh(core_axis_name="c", subcore_axis_name="s",
                                  num_cores=NC, num_subcores=NS)
    f = pl.kernel(
        _gcn_body,
        out_type=jax.ShapeDtypeStruct((NC, N, F), _f32),
        mesh=mesh,
        compiler_params=pltpu.CompilerParams(needs_layout_passes=False),
        scratch_types=[
            pltpu.VMEM((EPT,), _i32),
            pltpu.VMEM((EPT,), _i32),
            pltpu.VMEM((K2, F), _f32),
            pltpu.VMEM((2, K2), _i32),
            pltpu.VMEM((16, 16), _f32),
            pltpu.VMEM_SHARED((N, F), _f32),
            pltpu.SemaphoreType.DMA,
        ],
    )
    return f(src, dst, q)


def _e_body(a0_ref, a1_ref, db_ref, lt_ref, b2_ref, o_ref):
    acc = a0_ref[...] + a1_ref[...]
    v = (db_ref[...] * acc[:, :C]
         + lt_ref[...] + b2_ref[...])
    m = jnp.max(v, axis=1, keepdims=True)
    lse = jnp.log(jnp.sum(jnp.exp(v - m), axis=1, keepdims=True)) + m
    o_ref[...] = v - lse


def _run_e(a0, a1, db, lt, b2):
    bn = 1000
    return pl.pallas_call(
        _e_body,
        grid=(N // bn,),
        in_specs=[
            pl.BlockSpec((bn, F), lambda i: (i, 0)),
            pl.BlockSpec((bn, F), lambda i: (i, 0)),
            pl.BlockSpec((bn, C), lambda i: (i, 0)),
            pl.BlockSpec((bn, C), lambda i: (i, 0)),
            pl.BlockSpec((1, C), lambda i: (0, 0)),
        ],
        out_specs=pl.BlockSpec((bn, C), lambda i: (i, 0)),
        out_shape=jax.ShapeDtypeStruct((N, C), _f32),
    )(a0, a1, db, lt, b2.reshape(1, C))


def kernel(x, edge_index, Wl, bl, att, gat_bias, W2, b2):
    att_flat = att.reshape(HF)
    src = edge_index[0]
    dst = edge_index[1]
    xls = _run_a(x, Wl, bl)
    gat_part, s_part = _run_gat(src, dst, xls, att_flat)
    gps = [gat_part[c, h, :N] for c in range(NC) for h in range(H)]
    q, lt, db = _run_c(xls, gps, s_part[0], s_part[1], att, gat_bias, W2)
    out_part = _run_gcn(src, dst, q)
    return _run_e(out_part[0], out_part[1], db, lt, b2)
